# baseline (speedup 1.0000x reference)
import functools

import jax
import jax.numpy as jnp
from jax import lax
from jax.experimental import pallas as pl
from jax.experimental.pallas import tpu as pltpu
from jax.experimental.pallas import tpu_sc as plsc

_VOCAB = 30522
_H = 768
_B = 64
_S = 512
_EPS = 1e-5

_NC = 2
_NS = 16
_NW = _NC * _NS
_PPW = _S // _NW
_L = 16
_NVR = _H // _L
_NP = _B // 2


def _body(ids_hbm, ww_hbm, wp_hbm, wt_hbm, g_hbm, bt_hbm, out_hbm,
          ids_v, pos_v, wt_v, g_v, bt_v,
          rows0a, rows0b, rows1a, rows1b,
          sums_a, sq_a, sums_b, sq_b,
          gsem0, gsem1, osem0, osem1):
    wid = lax.axis_index("s") * _NC + lax.axis_index("c")
    p0 = wid * _PPW

    pltpu.sync_copy(ids_hbm, ids_v)
    pltpu.sync_copy(wp_hbm.at[pl.ds(p0, _PPW)], pos_v)
    pltpu.sync_copy(wt_hbm, wt_v)
    pltpu.sync_copy(g_hbm, g_v)
    pltpu.sync_copy(bt_hbm, bt_v)

    @plsc.parallel_loop(0, _PPW * _NVR)
    def _fold(i):
        t = i // _NVR
        sl = pl.ds((i % _NVR) * _L, _L)
        pos_v[t, sl] = pos_v[t, sl] + wt_v[0, sl]

    inv_h = 1.0 / _H
    lane_iota = lax.iota(jnp.int32, _L)
    zero = jnp.zeros((_L,), jnp.float32)

    def start_gathers(p, bufa, bufb, sem):
        idxa = ids_v[2 * p, pl.ds(p0, _PPW)]
        pltpu.async_copy(ww_hbm.at[idxa], bufa, sem)
        idxb = ids_v[2 * p + 1, pl.ds(p0, _PPW)]
        pltpu.async_copy(ww_hbm.at[idxb], bufb, sem)

    def wait_gathers(bufa, bufb, sem):
        pltpu.make_async_copy(ww_hbm.at[pl.ds(0, _PPW)], bufa, sem).wait()
        pltpu.make_async_copy(ww_hbm.at[pl.ds(0, _PPW)], bufb, sem).wait()

    def start_outs(p, bufa, bufb, sem):
        pltpu.async_copy(bufa, out_hbm.at[pl.ds(2 * p * _S + p0, _PPW)], sem)
        pltpu.async_copy(bufb, out_hbm.at[pl.ds((2 * p + 1) * _S + p0, _PPW)], sem)

    def wait_outs(bufa, bufb, sem):
        pltpu.make_async_copy(bufa, out_hbm.at[pl.ds(0, _PPW)], sem).wait()
        pltpu.make_async_copy(bufb, out_hbm.at[pl.ds(0, _PPW)], sem).wait()

    def ln_scale(sums_ref, sq_ref):
        t1 = [zero] * 4
        t2 = [zero] * 4
        for i in range(_L):
            sl = pl.ds(0, _L)
            t1[i % 4] = t1[i % 4] + sums_ref[i, sl]
            t2[i % 4] = t2[i % 4] + sq_ref[i, sl]
        tot = (t1[0] + t1[1]) + (t1[2] + t1[3])
        tot2 = (t2[0] + t2[1]) + (t2[2] + t2[3])
        mean_vec = tot * inv_h
        var = tot2 * inv_h - mean_vec * mean_vec + _EPS
        iv = lax.bitcast_convert_type(var, jnp.int32)
        iv = 0x5F3759DF - lax.shift_right_logical(iv, 1)
        y = lax.bitcast_convert_type(iv, jnp.float32)
        y = y * (1.5 - 0.5 * var * y * y)
        y = y * (1.5 - 0.5 * var * y * y)
        y = y * (1.5 - 0.5 * var * y * y)
        return mean_vec, y

    def compute(bufa, bufb):
        @plsc.parallel_loop(0, _PPW)
        def p1(t):
            col = jnp.broadcast_to(t, (_L,))
            acca = [zero] * 2
            acc2a = [zero] * 2
            accb = [zero] * 2
            acc2b = [zero] * 2
            for j in range(_NVR):
                sl = pl.ds(j * _L, _L)
                q = pos_v[t, sl]
                xa = bufa[t, sl] + q
                bufa[t, sl] = xa
                xb = bufb[t, sl] + q
                bufb[t, sl] = xb
                acca[j % 2] = acca[j % 2] + xa
                acc2a[j % 2] = acc2a[j % 2] + xa * xa
                accb[j % 2] = accb[j % 2] + xb
                acc2b[j % 2] = acc2b[j % 2] + xb * xb
            plsc.store_scatter(sums_a, [lane_iota, col], acca[0] + acca[1])
            plsc.store_scatter(sq_a, [lane_iota, col], acc2a[0] + acc2a[1])
            plsc.store_scatter(sums_b, [lane_iota, col], accb[0] + accb[1])
            plsc.store_scatter(sq_b, [lane_iota, col], acc2b[0] + acc2b[1])

        mean_a, rstd_a = ln_scale(sums_a, sq_a)
        mean_b, rstd_b = ln_scale(sums_b, sq_b)

        for buf, mean_vec, rstd_vec in ((bufa, mean_a, rstd_a),
                                        (bufb, mean_b, rstd_b)):
            means = [jnp.broadcast_to(mean_vec[t], (_L,)) for t in range(_PPW)]
            rstds = [jnp.broadcast_to(rstd_vec[t], (_L,)) for t in range(_PPW)]

            @plsc.parallel_loop(0, _NVR)
            def p2(j, buf=buf, means=means, rstds=rstds):
                sl = pl.ds(j * _L, _L)
                g = g_v[sl]
                bt = bt_v[sl]
                for t in range(_PPW):
                    x = buf[t, sl]
                    buf[t, sl] = (x - means[t]) * (rstds[t] * g) + bt

    slots = ((rows0a, rows0b, gsem0, osem0), (rows1a, rows1b, gsem1, osem1))
    start_gathers(0, rows0a, rows0b, gsem0)

    def outer(i2, _):
        for k in range(2):
            p = i2 * 2 + k
            bufa, bufb, gsem, osem = slots[k]
            obufa, obufb, ogsem, oosem = slots[1 - k]
            if k == 0:
                @pl.when(i2 > 0)
                def _():
                    wait_outs(obufa, obufb, oosem)
                start_gathers(p + 1, obufa, obufb, ogsem)
            else:
                wait_outs(obufa, obufb, oosem)

                @pl.when(i2 < _NP // 2 - 1)
                def _():
                    start_gathers(p + 1, obufa, obufb, ogsem)
            wait_gathers(bufa, bufb, gsem)
            compute(bufa, bufb)
            start_outs(p, bufa, bufb, osem)
        return 0

    lax.fori_loop(0, _NP // 2, outer, 0)
    wait_outs(rows1a, rows1b, osem1)


@jax.jit
def _launch(ids, ww, wp, wt, g, bt):
    mesh = plsc.VectorSubcoreMesh(core_axis_name="c", subcore_axis_name="s")
    run = functools.partial(
        pl.kernel,
        out_type=jax.ShapeDtypeStruct((_B * _S, _H), jnp.float32),
        mesh=mesh,
        compiler_params=pltpu.CompilerParams(needs_layout_passes=False),
        scratch_types=[
            pltpu.VMEM((_B, _S), jnp.int32),
            pltpu.VMEM((_PPW, _H), jnp.float32),
            pltpu.VMEM((2, _H), jnp.float32),
            pltpu.VMEM((_H,), jnp.float32),
            pltpu.VMEM((_H,), jnp.float32),
            pltpu.VMEM((_PPW, _H), jnp.float32),
            pltpu.VMEM((_PPW, _H), jnp.float32),
            pltpu.VMEM((_PPW, _H), jnp.float32),
            pltpu.VMEM((_PPW, _H), jnp.float32),
            pltpu.VMEM((_L, _PPW), jnp.float32),
            pltpu.VMEM((_L, _PPW), jnp.float32),
            pltpu.VMEM((_L, _PPW), jnp.float32),
            pltpu.VMEM((_L, _PPW), jnp.float32),
            pltpu.SemaphoreType.DMA,
            pltpu.SemaphoreType.DMA,
            pltpu.SemaphoreType.DMA,
            pltpu.SemaphoreType.DMA,
        ],
    )(_body)
    return run(ids, ww, wp, wt, g, bt)


def kernel(input_ids, W_word, W_pos, W_type, gamma, beta):
    ids = input_ids.astype(jnp.int32)
    out = _launch(ids, W_word, W_pos, W_type, gamma, beta)
    return out.reshape(_B, _S, _H)
